# baseline (speedup 1.0000x reference)
import functools

import jax
import jax.numpy as jnp
from jax import lax
from jax.experimental import pallas as pl
from jax.experimental.pallas import tpu as pltpu
from jax.experimental.pallas import tpu_sc as plsc

N = 10000
NPAD = 10240
E = 320000
D = 128
NC = 2
NS = 16
NW = NC * NS
EPW = E // NW
CHUNK = 80
NCHUNK = EPW // CHUNK
RPS = NPAD // NS
DEGW = 16
CRPS = 624
CTAIL = N - NS * CRPS
DCH = 128

_MESH = plsc.VectorSubcoreMesh(core_axis_name="c", subcore_axis_name="s")


@functools.partial(
    pl.kernel,
    out_type=jax.ShapeDtypeStruct((NC, NPAD, DEGW), jnp.float32),
    mesh=_MESH,
    scratch_types=[
        pltpu.VMEM_SHARED((NPAD, DEGW), jnp.float32),
        pltpu.VMEM((EPW,), jnp.int32),
        pltpu.VMEM((DCH, DEGW), jnp.float32),
        pltpu.VMEM((RPS, DEGW), jnp.float32),
        pltpu.SemaphoreType.DMA,
    ],
)
def _sc_degree(dst_hbm, out_hbm, acc, dstb, ones, zbuf, sem):
    cid = lax.axis_index("c")
    sid = lax.axis_index("s")
    wid = cid * NS + sid

    def _fill(i, _):
        zbuf[i, pl.ds(0, 16)] = jnp.zeros((16,), jnp.float32)
        ones[i % DCH, pl.ds(0, 16)] = jnp.ones((16,), jnp.float32)
        return 0

    lax.fori_loop(0, RPS, _fill, 0)
    pltpu.sync_copy(zbuf, acc.at[pl.ds(sid * RPS, RPS)])
    pltpu.sync_copy(dst_hbm.at[pl.ds(wid * EPW, EPW)], dstb)
    plsc.subcore_barrier()

    WIN = 8
    NFULL = EPW // DCH
    DTAIL = EPW - NFULL * DCH

    def _dst(g):
        return acc.at[dstb.at[pl.ds(g * DCH, DCH)]]

    def _body(g, _):
        pltpu.async_copy(ones, _dst(g), sem, add=True)

        @pl.when(g >= WIN)
        def _():
            pltpu.make_async_copy(ones, _dst(g - WIN), sem).wait()

        return 0

    lax.fori_loop(0, NFULL, _body, 0)
    pltpu.async_copy(
        ones.at[pl.ds(0, DTAIL)],
        acc.at[dstb.at[pl.ds(NFULL * DCH, DTAIL)]], sem, add=True)
    for w in range(WIN):
        pltpu.make_async_copy(ones, _dst(NFULL - WIN + w), sem).wait()
    pltpu.make_async_copy(
        ones.at[pl.ds(0, DTAIL)],
        acc.at[dstb.at[pl.ds(NFULL * DCH, DTAIL)]], sem).wait()
    plsc.subcore_barrier()
    pltpu.sync_copy(acc.at[pl.ds(sid * RPS, RPS)],
                    out_hbm.at[cid, pl.ds(sid * RPS, RPS)])


@functools.partial(
    pl.kernel,
    out_type=jax.ShapeDtypeStruct((NC, N, D), jnp.float32),
    mesh=_MESH,
    scratch_types=[
        pltpu.VMEM_SHARED((N, D), jnp.float32),
        pltpu.VMEM((EPW,), jnp.int32),
        pltpu.VMEM((EPW,), jnp.int32),
        pltpu.VMEM((CHUNK, D), jnp.float32),
        pltpu.VMEM((CHUNK, D), jnp.float32),
        pltpu.VMEM((CHUNK, D), jnp.float32),
        pltpu.SemaphoreType.DMA,
        pltpu.SemaphoreType.DMA,
        pltpu.SemaphoreType.DMA,
        pltpu.SemaphoreType.DMA,
        pltpu.SemaphoreType.DMA,
        pltpu.SemaphoreType.DMA,
        pltpu.SemaphoreType.DMA,
    ],
)
def _sc_aggregate(src_hbm, dst_hbm, hs_hbm, zeros_hbm, out_hbm,
                  acc, srcb, dstb, rows0, rows1, rows2,
                  semg0, semg1, semg2, sems0, sems1, sems2, semz):
    cid = lax.axis_index("c")
    sid = lax.axis_index("s")
    wid = cid * NS + sid
    rows = (rows0, rows1, rows2)
    gsem = (semg0, semg1, semg2)
    ssem = (sems0, sems1, sems2)

    pltpu.async_copy(zeros_hbm.at[pl.ds(sid * CRPS, CRPS)],
                     acc.at[pl.ds(sid * CRPS, CRPS)], semz)

    @pl.when(sid == 0)
    def _():
        pltpu.async_copy(zeros_hbm.at[pl.ds(NS * CRPS, CTAIL)],
                         acc.at[pl.ds(NS * CRPS, CTAIL)], semz)

    pltpu.async_copy(src_hbm.at[pl.ds(wid * EPW, EPW)], srcb, semg0)
    pltpu.async_copy(dst_hbm.at[pl.ds(wid * EPW, EPW)], dstb, semg1)
    pltpu.make_async_copy(zeros_hbm.at[pl.ds(sid * CRPS, CRPS)],
                          acc.at[pl.ds(sid * CRPS, CRPS)], semz).wait()

    @pl.when(sid == 0)
    def _():
        pltpu.make_async_copy(zeros_hbm.at[pl.ds(NS * CRPS, CTAIL)],
                              acc.at[pl.ds(NS * CRPS, CTAIL)], semz).wait()

    pltpu.make_async_copy(src_hbm.at[pl.ds(wid * EPW, EPW)], srcb,
                          semg0).wait()
    pltpu.make_async_copy(dst_hbm.at[pl.ds(wid * EPW, EPW)], dstb,
                          semg1).wait()
    plsc.subcore_barrier()

    def _gref(g):
        return hs_hbm.at[srcb.at[pl.ds(g * CHUNK, CHUNK)]]

    def _sref(g):
        return acc.at[dstb.at[pl.ds(g * CHUNK, CHUNK)]]

    def _gather(g, b):
        pltpu.async_copy(_gref(g), rows[b], gsem[b])

    def _gather_wait(g, b):
        pltpu.make_async_copy(_gref(g), rows[b], gsem[b]).wait()

    def _scatter(g, b):
        pltpu.async_copy(rows[b], _sref(g), ssem[b], add=True)

    def _scatter_wait(g, b):
        pltpu.make_async_copy(rows[b], _sref(g), ssem[b]).wait()

    _gather(0, 0)
    _gather(1, 1)
    _gather_wait(0, 0)
    _scatter(0, 0)
    _gather(2, 2)
    _gather_wait(1, 1)
    _scatter(1, 1)

    def _body(i, _):
        for k in range(3):
            g = 3 * i + 2 + k
            b = (2 + k) % 3
            _scatter_wait(g - 2, k)

            @pl.when(g + 1 < NCHUNK)
            def _():
                _gather(g + 1, k)

            _gather_wait(g, b)
            _scatter(g, b)
        return 0

    assert (NCHUNK - 2) % 3 == 0
    lax.fori_loop(0, (NCHUNK - 2) // 3, _body, 0)
    _scatter_wait(NCHUNK - 2, (NCHUNK - 2) % 3)
    _scatter_wait(NCHUNK - 1, (NCHUNK - 1) % 3)
    plsc.subcore_barrier()
    pltpu.async_copy(acc.at[pl.ds(sid * CRPS, CRPS)],
                     out_hbm.at[cid, pl.ds(sid * CRPS, CRPS)], semz)

    @pl.when(sid == 0)
    def _():
        pltpu.async_copy(acc.at[pl.ds(NS * CRPS, CTAIL)],
                         out_hbm.at[cid, pl.ds(NS * CRPS, CTAIL)], semz)

    pltpu.make_async_copy(acc.at[pl.ds(sid * CRPS, CRPS)],
                          out_hbm.at[cid, pl.ds(sid * CRPS, CRPS)],
                          semz).wait()

    @pl.when(sid == 0)
    def _():
        pltpu.make_async_copy(acc.at[pl.ds(NS * CRPS, CTAIL)],
                              out_hbm.at[cid, pl.ds(NS * CRPS, CTAIL)],
                              semz).wait()


BLK = 1000
GRID = N // BLK


def _k1_body(x_ref, w_ref, degp_ref, hs_ref, dinv_ref):
    deg = degp_ref[0, :, 0:1] + degp_ref[1, :, 0:1]
    dinv = lax.rsqrt(deg)
    y = jnp.dot(x_ref[...], w_ref[...], preferred_element_type=jnp.float32)
    hs_ref[...] = y * dinv
    dinv_ref[...] = dinv


def _k2_body(p_ref, hs1_ref, dinv_ref, b1_ref, w2_ref, hs2_ref):
    dinv = dinv_ref[...]
    agg = p_ref[0] + p_ref[1] + hs1_ref[...]
    h = jnp.maximum(agg * dinv + b1_ref[...], 0.0)
    y2 = jnp.dot(h, w2_ref[...], preferred_element_type=jnp.float32)
    hs2_ref[...] = y2 * dinv


def _k3_body(q_ref, hs2_ref, dinv_ref, b2_ref, out_ref):
    agg = q_ref[0] + q_ref[1] + hs2_ref[...]
    out_ref[...] = agg * dinv_ref[...] + b2_ref[...]


def _row_spec(shape):
    nd = len(shape)
    if nd == 2:
        return pl.BlockSpec((BLK, shape[1]), lambda i: (i, 0))
    return pl.BlockSpec((shape[0], BLK, shape[2]), lambda i: (0, i, 0))


def _full_spec(shape):
    return pl.BlockSpec(shape, lambda i: tuple(0 for _ in shape))


def kernel(x, edge_index, W1, b1, W2, b2):
    src = edge_index[0].astype(jnp.int32)
    dst = edge_index[1].astype(jnp.int32)
    b1r = b1.reshape(1, D)
    b2r = b2.reshape(1, D)
    zeros = jnp.zeros((N, D), jnp.float32)

    degp = _sc_degree(dst)

    hs1, dinv = pl.pallas_call(
        _k1_body,
        grid=(GRID,),
        in_specs=[_row_spec((N, D)), _full_spec((D, D)),
                  _row_spec((NC, NPAD, DEGW))],
        out_specs=[_row_spec((N, D)),
                   pl.BlockSpec((BLK, 1), lambda i: (i, 0))],
        out_shape=[jax.ShapeDtypeStruct((N, D), jnp.float32),
                   jax.ShapeDtypeStruct((N, 1), jnp.float32)],
    )(x, W1, degp)

    p1 = _sc_aggregate(src, dst, hs1, zeros)

    hs2 = pl.pallas_call(
        _k2_body,
        grid=(GRID,),
        in_specs=[_row_spec((NC, N, D)), _row_spec((N, D)),
                  pl.BlockSpec((BLK, 1), lambda i: (i, 0)),
                  _full_spec((1, D)), _full_spec((D, D))],
        out_specs=_row_spec((N, D)),
        out_shape=jax.ShapeDtypeStruct((N, D), jnp.float32),
    )(p1, hs1, dinv, b1r, W2)

    p2 = _sc_aggregate(src, dst, hs2, zeros)

    logits = pl.pallas_call(
        _k3_body,
        grid=(GRID,),
        in_specs=[_row_spec((NC, N, D)), _row_spec((N, D)),
                  pl.BlockSpec((BLK, 1), lambda i: (i, 0)),
                  _full_spec((1, D))],
        out_specs=_row_spec((N, D)),
        out_shape=jax.ShapeDtypeStruct((N, D), jnp.float32),
    )(p2, hs2, dinv, b2r)

    return logits


# speedup vs baseline: 1.0058x; 1.0058x over previous
import functools

import jax
import jax.numpy as jnp
from jax import lax
from jax.experimental import pallas as pl
from jax.experimental.pallas import tpu as pltpu
from jax.experimental.pallas import tpu_sc as plsc

N = 10000
NPAD = 10240
E = 320000
D = 128
NC = 2
NS = 16
NW = NC * NS
EPW = E // NW
CHUNK = 80
NCHUNK = EPW // CHUNK
RPS = NPAD // NS
DEGW = 16
CRPS = 624
CTAIL = N - NS * CRPS
DCH = CHUNK

_MESH = plsc.VectorSubcoreMesh(core_axis_name="c", subcore_axis_name="s")


@functools.partial(
    pl.kernel,
    out_type=jax.ShapeDtypeStruct((NC, NPAD, DEGW), jnp.float32),
    mesh=_MESH,
    scratch_types=[
        pltpu.VMEM_SHARED((NPAD, DEGW), jnp.float32),
        pltpu.VMEM((EPW,), jnp.int32),
        pltpu.VMEM((DCH, DEGW), jnp.float32),
        pltpu.VMEM((RPS, DEGW), jnp.float32),
        pltpu.SemaphoreType.DMA,
    ],
)
def _sc_degree(dst_hbm, out_hbm, acc, dstb, ones, zbuf, sem):
    cid = lax.axis_index("c")
    sid = lax.axis_index("s")
    wid = cid * NS + sid

    def _fill(i, _):
        zbuf[i, pl.ds(0, 16)] = jnp.zeros((16,), jnp.float32)
        ones[i % DCH, pl.ds(0, 16)] = jnp.ones((16,), jnp.float32)
        return 0

    lax.fori_loop(0, RPS, _fill, 0)
    pltpu.sync_copy(zbuf, acc.at[pl.ds(sid * RPS, RPS)])
    pltpu.sync_copy(dst_hbm.at[pl.ds(wid * EPW, EPW)], dstb)
    plsc.subcore_barrier()

    WIN = 8

    def _dst(g):
        return acc.at[dstb.at[pl.ds(g * DCH, DCH)]]

    def _body(g, _):
        pltpu.async_copy(ones, _dst(g), sem, add=True)

        @pl.when(g >= WIN)
        def _():
            pltpu.make_async_copy(ones, _dst(g - WIN), sem).wait()

        return 0

    lax.fori_loop(0, NCHUNK, _body, 0)
    for w in range(WIN):
        pltpu.make_async_copy(ones, _dst(NCHUNK - WIN + w), sem).wait()
    plsc.subcore_barrier()
    pltpu.sync_copy(acc.at[pl.ds(sid * RPS, RPS)],
                    out_hbm.at[cid, pl.ds(sid * RPS, RPS)])


@functools.partial(
    pl.kernel,
    out_type=jax.ShapeDtypeStruct((NC, N, D), jnp.float32),
    mesh=_MESH,
    scratch_types=[
        pltpu.VMEM_SHARED((N, D), jnp.float32),
        pltpu.VMEM((EPW,), jnp.int32),
        pltpu.VMEM((EPW,), jnp.int32),
        pltpu.VMEM((CHUNK, D), jnp.float32),
        pltpu.VMEM((CHUNK, D), jnp.float32),
        pltpu.VMEM((CHUNK, D), jnp.float32),
        pltpu.SemaphoreType.DMA,
        pltpu.SemaphoreType.DMA,
        pltpu.SemaphoreType.DMA,
        pltpu.SemaphoreType.DMA,
        pltpu.SemaphoreType.DMA,
        pltpu.SemaphoreType.DMA,
        pltpu.SemaphoreType.DMA,
    ],
)
def _sc_aggregate(src_hbm, dst_hbm, hs_hbm, zeros_hbm, out_hbm,
                  acc, srcb, dstb, rows0, rows1, rows2,
                  semg0, semg1, semg2, sems0, sems1, sems2, semz):
    cid = lax.axis_index("c")
    sid = lax.axis_index("s")
    wid = cid * NS + sid
    rows = (rows0, rows1, rows2)
    gsem = (semg0, semg1, semg2)
    ssem = (sems0, sems1, sems2)

    pltpu.async_copy(zeros_hbm.at[pl.ds(sid * CRPS, CRPS)],
                     acc.at[pl.ds(sid * CRPS, CRPS)], semz)

    @pl.when(sid == 0)
    def _():
        pltpu.async_copy(zeros_hbm.at[pl.ds(NS * CRPS, CTAIL)],
                         acc.at[pl.ds(NS * CRPS, CTAIL)], semz)

    pltpu.async_copy(src_hbm.at[pl.ds(wid * EPW, EPW)], srcb, semg0)
    pltpu.async_copy(dst_hbm.at[pl.ds(wid * EPW, EPW)], dstb, semg1)
    pltpu.make_async_copy(zeros_hbm.at[pl.ds(sid * CRPS, CRPS)],
                          acc.at[pl.ds(sid * CRPS, CRPS)], semz).wait()

    @pl.when(sid == 0)
    def _():
        pltpu.make_async_copy(zeros_hbm.at[pl.ds(NS * CRPS, CTAIL)],
                              acc.at[pl.ds(NS * CRPS, CTAIL)], semz).wait()

    pltpu.make_async_copy(src_hbm.at[pl.ds(wid * EPW, EPW)], srcb,
                          semg0).wait()
    pltpu.make_async_copy(dst_hbm.at[pl.ds(wid * EPW, EPW)], dstb,
                          semg1).wait()
    plsc.subcore_barrier()

    def _gref(g):
        return hs_hbm.at[srcb.at[pl.ds(g * CHUNK, CHUNK)]]

    def _sref(g):
        return acc.at[dstb.at[pl.ds(g * CHUNK, CHUNK)]]

    def _gather(g, b):
        pltpu.async_copy(_gref(g), rows[b], gsem[b])

    def _gather_wait(g, b):
        pltpu.make_async_copy(_gref(g), rows[b], gsem[b]).wait()

    def _scatter(g, b):
        pltpu.async_copy(rows[b], _sref(g), ssem[b], add=True)

    def _scatter_wait(g, b):
        pltpu.make_async_copy(rows[b], _sref(g), ssem[b]).wait()

    _gather(0, 0)
    _gather(1, 1)
    _gather_wait(0, 0)
    _scatter(0, 0)
    _gather(2, 2)
    _gather_wait(1, 1)
    _scatter(1, 1)

    def _body(i, _):
        for k in range(3):
            g = 3 * i + 2 + k
            b = (2 + k) % 3
            _scatter_wait(g - 2, k)

            @pl.when(g + 1 < NCHUNK)
            def _():
                _gather(g + 1, k)

            _gather_wait(g, b)
            _scatter(g, b)
        return 0

    assert (NCHUNK - 2) % 3 == 0
    lax.fori_loop(0, (NCHUNK - 2) // 3, _body, 0)
    _scatter_wait(NCHUNK - 2, (NCHUNK - 2) % 3)
    _scatter_wait(NCHUNK - 1, (NCHUNK - 1) % 3)
    plsc.subcore_barrier()
    pltpu.async_copy(acc.at[pl.ds(sid * CRPS, CRPS)],
                     out_hbm.at[cid, pl.ds(sid * CRPS, CRPS)], semz)

    @pl.when(sid == 0)
    def _():
        pltpu.async_copy(acc.at[pl.ds(NS * CRPS, CTAIL)],
                         out_hbm.at[cid, pl.ds(NS * CRPS, CTAIL)], semz)

    pltpu.make_async_copy(acc.at[pl.ds(sid * CRPS, CRPS)],
                          out_hbm.at[cid, pl.ds(sid * CRPS, CRPS)],
                          semz).wait()

    @pl.when(sid == 0)
    def _():
        pltpu.make_async_copy(acc.at[pl.ds(NS * CRPS, CTAIL)],
                              out_hbm.at[cid, pl.ds(NS * CRPS, CTAIL)],
                              semz).wait()


BLK = 1000
GRID = N // BLK


def _k1_body(x_ref, w_ref, degp_ref, hs_ref, dinv_ref):
    deg = degp_ref[0, :, 0:1] + degp_ref[1, :, 0:1]
    dinv = lax.rsqrt(deg)
    y = jnp.dot(x_ref[...], w_ref[...], preferred_element_type=jnp.float32)
    hs_ref[...] = y * dinv
    dinv_ref[...] = dinv


def _k2_body(p_ref, hs1_ref, dinv_ref, b1_ref, w2_ref, hs2_ref):
    dinv = dinv_ref[...]
    agg = p_ref[0] + p_ref[1] + hs1_ref[...]
    h = jnp.maximum(agg * dinv + b1_ref[...], 0.0)
    y2 = jnp.dot(h, w2_ref[...], preferred_element_type=jnp.float32)
    hs2_ref[...] = y2 * dinv


def _k3_body(q_ref, hs2_ref, dinv_ref, b2_ref, out_ref):
    agg = q_ref[0] + q_ref[1] + hs2_ref[...]
    out_ref[...] = agg * dinv_ref[...] + b2_ref[...]


def _row_spec(shape):
    nd = len(shape)
    if nd == 2:
        return pl.BlockSpec((BLK, shape[1]), lambda i: (i, 0))
    return pl.BlockSpec((shape[0], BLK, shape[2]), lambda i: (0, i, 0))


def _full_spec(shape):
    return pl.BlockSpec(shape, lambda i: tuple(0 for _ in shape))


def kernel(x, edge_index, W1, b1, W2, b2):
    src = edge_index[0].astype(jnp.int32)
    dst = edge_index[1].astype(jnp.int32)
    b1r = b1.reshape(1, D)
    b2r = b2.reshape(1, D)
    zeros = jnp.zeros((N, D), jnp.float32)

    degp = _sc_degree(dst)

    hs1, dinv = pl.pallas_call(
        _k1_body,
        grid=(GRID,),
        in_specs=[_row_spec((N, D)), _full_spec((D, D)),
                  _row_spec((NC, NPAD, DEGW))],
        out_specs=[_row_spec((N, D)),
                   pl.BlockSpec((BLK, 1), lambda i: (i, 0))],
        out_shape=[jax.ShapeDtypeStruct((N, D), jnp.float32),
                   jax.ShapeDtypeStruct((N, 1), jnp.float32)],
    )(x, W1, degp)

    p1 = _sc_aggregate(src, dst, hs1, zeros)

    hs2 = pl.pallas_call(
        _k2_body,
        grid=(GRID,),
        in_specs=[_row_spec((NC, N, D)), _row_spec((N, D)),
                  pl.BlockSpec((BLK, 1), lambda i: (i, 0)),
                  _full_spec((1, D)), _full_spec((D, D))],
        out_specs=_row_spec((N, D)),
        out_shape=jax.ShapeDtypeStruct((N, D), jnp.float32),
    )(p1, hs1, dinv, b1r, W2)

    p2 = _sc_aggregate(src, dst, hs2, zeros)

    logits = pl.pallas_call(
        _k3_body,
        grid=(GRID,),
        in_specs=[_row_spec((NC, N, D)), _row_spec((N, D)),
                  pl.BlockSpec((BLK, 1), lambda i: (i, 0)),
                  _full_spec((1, D))],
        out_specs=_row_spec((N, D)),
        out_shape=jax.ShapeDtypeStruct((N, D), jnp.float32),
    )(p2, hs2, dinv, b2r)

    return logits
